# baseline (speedup 1.0000x reference)
import functools

import jax
import jax.numpy as jnp
from jax import lax
from jax.experimental import pallas as pl
from jax.experimental.pallas import tpu as pltpu
from jax.experimental.pallas import tpu_sc as plsc

B, L, D, V = 512, 512, 128, 1024
PAD_ID, EOS_ID = 0, 1
N = B * L
NC, NS = 2, 16
NW = NC * NS
SLOT = 64
NBUF = 8
SPW = N // (NW * SLOT)
IPW = N // NW

_mesh = plsc.VectorSubcoreMesh(core_axis_name="c", subcore_axis_name="s")


@functools.partial(
    pl.kernel,
    out_type=jax.ShapeDtypeStruct((N, D), jnp.float32),
    mesh=_mesh,
    scratch_types=[
        pltpu.VMEM((IPW,), jnp.int32),
        pltpu.VMEM((NBUF, SLOT, D), jnp.float32),
        pltpu.VMEM_SHARED((V, D), jnp.float32),
        [pltpu.SemaphoreType.DMA] * NBUF,
        [pltpu.SemaphoreType.DMA] * NBUF,
    ],
)
def _gather_sc(ids_hbm, table_hbm, out_hbm, idx_v, rows_v, tab_sh, gsem, osem):
    wid = lax.axis_index("s") * NC + lax.axis_index("c")
    base_id = wid * IPW

    @pl.when(lax.axis_index("s") == 0)
    def _():
        pltpu.sync_copy(table_hbm, tab_sh)

    pltpu.sync_copy(ids_hbm.at[pl.ds(base_id, IPW)], idx_v)
    plsc.subcore_barrier()

    use_hbm = lax.rem(lax.axis_index("s"), 2) == 1

    def gather(j, b):
        idxs = idx_v.at[pl.ds(j * SLOT, SLOT)]

        @pl.when(use_hbm)
        def _():
            pltpu.async_copy(table_hbm.at[idxs], rows_v.at[b], gsem[b])

        @pl.when(jnp.logical_not(use_hbm))
        def _():
            pltpu.async_copy(tab_sh.at[idxs], rows_v.at[b], gsem[b])

    def wait_gather(b):
        pltpu.make_async_copy(
            tab_sh.at[idx_v.at[pl.ds(0, SLOT)]], rows_v.at[b], gsem[b]).wait()

    def put(j, b):
        pltpu.async_copy(
            rows_v.at[b], out_hbm.at[pl.ds(base_id + j * SLOT, SLOT)], osem[b])

    def wait_put(b):
        pltpu.make_async_copy(
            rows_v.at[b], out_hbm.at[pl.ds(0, SLOT)], osem[b]).wait()

    for b in range(NBUF):
        gather(b, b)

    def body(i, _):
        j0 = NBUF * i
        for b in range(NBUF):
            wait_gather(b)
            put(j0 + b, b)

        @pl.when(i + 1 < SPW // NBUF)
        def _():
            for b in range(NBUF):
                wait_put(b)
                gather(j0 + NBUF + b, b)

        return 0

    lax.fori_loop(0, SPW // NBUF, body, 0)
    for b in range(NBUF):
        wait_put(b)


def _mask_eos_body(ids_ref, mask_ref, eos_ref):
    ids = ids_ref[...]
    mask_ref[...] = ids == PAD_ID
    eos_ref[...] = (ids == EOS_ID).astype(jnp.float32)


_mask_eos = pl.pallas_call(
    _mask_eos_body,
    out_shape=(
        jax.ShapeDtypeStruct((B, L), jnp.bool_),
        jax.ShapeDtypeStruct((B, L), jnp.float32),
    ),
)


def kernel(lookup_ids, embedding_matrix):
    ids_flat = lookup_ids.reshape(N)
    matrices = _gather_sc(ids_flat, embedding_matrix).reshape(B, L, D)
    mask, eos = _mask_eos(lookup_ids)
    return (matrices, mask, eos, embedding_matrix, lookup_ids)


# speedup vs baseline: 1.5674x; 1.5674x over previous
import functools

import jax
import jax.numpy as jnp
from jax import lax
from jax.experimental import pallas as pl
from jax.experimental.pallas import tpu as pltpu
from jax.experimental.pallas import tpu_sc as plsc

B, L, D, V = 512, 512, 128, 1024
PAD_ID, EOS_ID = 0, 1
N = B * L
NC, NS = 2, 16
NW = NC * NS
SLOT = 64
NBUF = 8
SPW = N // (NW * SLOT)
IPW = N // NW

_mesh = plsc.VectorSubcoreMesh(core_axis_name="c", subcore_axis_name="s")


@functools.partial(
    pl.kernel,
    out_type=jax.ShapeDtypeStruct((N, D), jnp.float32),
    mesh=_mesh,
    scratch_types=[
        pltpu.VMEM((IPW,), jnp.int32),
        pltpu.VMEM((NBUF, SLOT, D), jnp.float32),
        pltpu.VMEM_SHARED((V, D), jnp.float32),
        [pltpu.SemaphoreType.DMA] * NBUF,
        [pltpu.SemaphoreType.DMA] * NBUF,
    ],
)
def _gather_sc(ids_hbm, table_hbm, out_hbm, idx_v, rows_v, tab_sh, gsem, osem):
    wid = lax.axis_index("s") * NC + lax.axis_index("c")
    base_id = wid * IPW

    @pl.when(lax.axis_index("s") == 0)
    def _():
        pltpu.sync_copy(table_hbm, tab_sh)

    pltpu.sync_copy(ids_hbm.at[pl.ds(base_id, IPW)], idx_v)
    plsc.subcore_barrier()

    def gather(j, b):
        pltpu.async_copy(
            tab_sh.at[idx_v.at[pl.ds(j * SLOT, SLOT)]], rows_v.at[b], gsem[b])

    def wait_gather(b):
        pltpu.make_async_copy(
            tab_sh.at[idx_v.at[pl.ds(0, SLOT)]], rows_v.at[b], gsem[b]).wait()

    def put(j, b):
        pltpu.async_copy(
            rows_v.at[b], out_hbm.at[pl.ds(base_id + j * SLOT, SLOT)], osem[b])

    def wait_put(b):
        pltpu.make_async_copy(
            rows_v.at[b], out_hbm.at[pl.ds(0, SLOT)], osem[b]).wait()

    for b in range(NBUF):
        gather(b, b)

    def body(i, _):
        j0 = NBUF * i
        for b in range(NBUF):
            wait_gather(b)
            put(j0 + b, b)

        @pl.when(i + 1 < SPW // NBUF)
        def _():
            for b in range(NBUF):
                wait_put(b)
                gather(j0 + NBUF + b, b)

        return 0

    lax.fori_loop(0, SPW // NBUF, body, 0)
    for b in range(NBUF):
        wait_put(b)


def _mask_eos_body(ids_ref, mask_ref, eos_ref):
    ids = ids_ref[...]
    mask_ref[...] = ids == PAD_ID
    eos_ref[...] = (ids == EOS_ID).astype(jnp.float32)


_mask_eos = pl.pallas_call(
    _mask_eos_body,
    out_shape=(
        jax.ShapeDtypeStruct((B, L), jnp.bool_),
        jax.ShapeDtypeStruct((B, L), jnp.float32),
    ),
)


def kernel(lookup_ids, embedding_matrix):
    ids_flat = lookup_ids.reshape(N)
    matrices = _gather_sc(ids_flat, embedding_matrix).reshape(B, L, D)
    mask, eos = _mask_eos(lookup_ids)
    return (matrices, mask, eos, embedding_matrix, lookup_ids)
